# baseline (speedup 1.0000x reference)
import functools

import jax
import jax.numpy as jnp
from jax import lax
from jax.experimental import pallas as pl
from jax.experimental.pallas import tpu as pltpu
from jax.experimental.pallas import tpu_sc as plsc

_F32 = jnp.float32
_B = 125


def _node_chunks(N):
    r_main = ((N // 16 + 15) // 16) * 16
    r_last = N - 15 * r_main
    assert 0 < r_last <= r_main
    return r_main, r_last


def _chunk_copy(src, dst, s, r_main, r_last):
    @pl.when(s < 15)
    def _():
        pltpu.sync_copy(src.at[pl.ds(s * r_main, r_main)],
                        dst.at[pl.ds(s * r_main, r_main)])

    @pl.when(s == 15)
    def _():
        pltpu.sync_copy(src.at[pl.ds(15 * r_main, r_last)],
                        dst.at[pl.ds(15 * r_main, r_last)])


def _make_deg_kernel(N, E):
    EC = E // 32
    VR = EC // 16
    assert EC % 16 == 0
    r_main, r_last = _node_chunks(N)
    mesh = plsc.VectorSubcoreMesh(core_axis_name="c", subcore_axis_name="s")

    @functools.partial(
        pl.kernel,
        out_type=[jax.ShapeDtypeStruct((N,), _F32),
                  jax.ShapeDtypeStruct((N,), _F32)],
        mesh=mesh,
        scratch_types=[
            pltpu.VMEM((EC,), jnp.int32),
            pltpu.VMEM((N,), _F32),
            pltpu.VMEM((r_main,), _F32),
            pltpu.VMEM((r_main,), _F32),
            pltpu.VMEM_SHARED((16 * N,), _F32),
        ],
        compiler_params=pltpu.CompilerParams(needs_layout_passes=False),
    )
    def deg_kernel(dst1d, out0, out1, idxbuf, hist, tmp, accsum, spacc):
        c = lax.axis_index("c")
        s = lax.axis_index("s")
        w = c * 16 + s
        zero16 = jnp.zeros((16,), _F32)

        def fill_z(i, carry):
            hist[pl.ds(i * 16, 16)] = zero16
            return carry

        lax.fori_loop(0, N // 16, fill_z, 0)
        pltpu.sync_copy(dst1d.at[pl.ds(w * EC, EC)], idxbuf)

        def count(i, carry):
            idx = idxbuf[pl.ds(i * 16, 16)]
            cnt, last = plsc.scan_count(idx)
            plsc.addupdate_scatter(hist, [idx], cnt.astype(_F32), mask=last)
            return carry

        lax.fori_loop(0, VR, count, 0)
        pltpu.sync_copy(hist, spacc.at[pl.ds(s * N, N)])
        plsc.subcore_barrier()

        def reduce_write(r0, nr, out):
            pltpu.sync_copy(spacc.at[pl.ds(r0, nr)], accsum.at[pl.ds(0, nr)])

            def add_hist(j, carry):
                pltpu.sync_copy(spacc.at[pl.ds(j * N + r0, nr)],
                                tmp.at[pl.ds(0, nr)])

                def add_vec(v, carry2):
                    sl = pl.ds(v * 16, 16)
                    accsum[sl] = accsum[sl] + tmp[sl]
                    return carry2

                lax.fori_loop(0, nr // 16, add_vec, 0)
                return carry

            lax.fori_loop(1, 16, add_hist, 0)
            pltpu.sync_copy(accsum.at[pl.ds(0, nr)], out.at[pl.ds(r0, nr)])

        def write_core(out):
            @pl.when(s < 15)
            def _():
                reduce_write(s * r_main, r_main, out)

            @pl.when(s == 15)
            def _():
                reduce_write(15 * r_main, r_last, out)

        @pl.when(c == 0)
        def _():
            write_core(out0)

        @pl.when(c == 1)
        def _():
            write_core(out1)

    return deg_kernel


def _make_agg_kernel(N, E):
    n_rows = E // _B
    KB = n_rows // 16
    CH = 8
    assert KB % CH == 0 and KB * 16 == n_rows
    r_main, r_last = _node_chunks(N)
    mesh = plsc.VectorSubcoreMesh(core_axis_name="c", subcore_axis_name="s")

    @functools.partial(
        pl.kernel,
        out_type=[jax.ShapeDtypeStruct((N, 128), _F32),
                  jax.ShapeDtypeStruct((N, 128), _F32)],
        mesh=mesh,
        scratch_types=[
            pltpu.VMEM((CH, _B), jnp.int32),
            pltpu.VMEM((CH, _B), jnp.int32),
            pltpu.VMEM((_B, 128), _F32),
            pltpu.VMEM((_B, 128), _F32),
            pltpu.VMEM_SHARED((N, 128), _F32),
            pltpu.SemaphoreType.DMA,
            pltpu.SemaphoreType.DMA,
            pltpu.SemaphoreType.DMA,
            pltpu.SemaphoreType.DMA,
        ],
    )
    def agg_kernel(xs0, xs1, src2d, dst2d, out0, out1,
                   isrc, idst, buf0, buf1, acc, gsem0, gsem1, ssem0, ssem1):
        c = lax.axis_index("c")
        s = lax.axis_index("s")
        row0 = s * KB

        def run(xs, out):
            _chunk_copy(xs, acc, s, r_main, r_last)
            plsc.subcore_barrier()

            def chunk(t, carry):
                r = row0 + t * CH
                pltpu.sync_copy(src2d.at[pl.ds(r, CH)], isrc)
                pltpu.sync_copy(dst2d.at[pl.ds(r, CH)], idst)

                def pair(p, carry2):
                    @pl.when(p == 0)
                    def _():
                        pltpu.async_copy(xs.at[isrc.at[0]], buf0, gsem0)
                        pltpu.async_copy(xs.at[isrc.at[1]], buf1, gsem1)

                    pltpu.make_async_copy(xs.at[isrc.at[2 * p]], buf0,
                                          gsem0).wait()
                    pltpu.async_copy(buf0, acc.at[idst.at[2 * p]], ssem0,
                                     add=True)
                    pltpu.make_async_copy(xs.at[isrc.at[2 * p + 1]], buf1,
                                          gsem1).wait()
                    pltpu.async_copy(buf1, acc.at[idst.at[2 * p + 1]], ssem1,
                                     add=True)
                    pltpu.make_async_copy(buf0, acc.at[idst.at[2 * p]],
                                          ssem0).wait()

                    @pl.when(p < CH // 2 - 1)
                    def _():
                        pltpu.async_copy(xs.at[isrc.at[2 * p + 2]], buf0,
                                         gsem0)

                    pltpu.make_async_copy(buf1, acc.at[idst.at[2 * p + 1]],
                                          ssem1).wait()

                    @pl.when(p < CH // 2 - 1)
                    def _():
                        pltpu.async_copy(xs.at[isrc.at[2 * p + 3]], buf1,
                                         gsem1)

                    return carry2

                lax.fori_loop(0, CH // 2, pair, 0)
                return carry

            lax.fori_loop(0, KB // CH, chunk, 0)
            plsc.subcore_barrier()
            _chunk_copy(acc, out, s, r_main, r_last)

        @pl.when(c == 0)
        def _():
            run(xs0, out0)

        @pl.when(c == 1)
        def _():
            run(xs1, out1)

    return agg_kernel


def _dis_of(d0_blk, d1_blk):
    return lax.rsqrt(d0_blk[...] + d1_blk[...] + 1.0)


def _tc1_body(d0, d1, x, w1, hs0, hs1):
    dis = _dis_of(d0, d1)
    h = jnp.dot(x[...], w1[...], preferred_element_type=_F32) * dis
    m = h.shape[1] // 2
    hs0[...] = h[:, :m]
    hs1[...] = h[:, m:]


def _tc2_body(d0, d1, g0, g1, b1, w2, hs0, hs1):
    dis = _dis_of(d0, d1)
    a = jnp.concatenate([g0[...], g1[...]], axis=1) * dis
    h1 = jnp.maximum(a + b1[...], 0.0)
    h = jnp.dot(h1, w2[...], preferred_element_type=_F32) * dis
    m = h.shape[1] // 2
    hs0[...] = h[:, :m]
    hs1[...] = h[:, m:]


def _tc3_body(d0, d1, a0, a1, b2, wh, bh, out):
    dis = _dis_of(d0, d1)
    a = jnp.concatenate([a0[...], a1[...]], axis=1) * dis
    h2 = jnp.maximum(a + b2[...], 0.0)
    out[...] = jnp.dot(h2, wh[...], preferred_element_type=_F32) + bh[...]


def _row_spec(blk, cols):
    return pl.BlockSpec((blk, cols), lambda i: (i, 0))


def _full_spec(shape):
    return pl.BlockSpec(shape, lambda i: (0,) * len(shape))


def kernel(x, edge_index, W1, b1, W2, b2, Wh, bh):
    N, D = x.shape
    H = W1.shape[1]
    E = edge_index.shape[1]
    F2 = H // 2
    BLK = 1000
    grid = (N // BLK,)

    src1d = edge_index[0].astype(jnp.int32)
    dst1d = edge_index[1].astype(jnp.int32)
    src2d = src1d.reshape(E // _B, _B)
    dst2d = dst1d.reshape(E // _B, _B)

    p0, p1 = _make_deg_kernel(N, E)(dst1d)
    d0 = p0.reshape(N, 1)
    d1 = p1.reshape(N, 1)

    agg = _make_agg_kernel(N, E)

    hs0, hs1 = pl.pallas_call(
        _tc1_body,
        grid=grid,
        in_specs=[_row_spec(BLK, 1), _row_spec(BLK, 1), _row_spec(BLK, D),
                  _full_spec((D, H))],
        out_specs=[_row_spec(BLK, F2), _row_spec(BLK, F2)],
        out_shape=[jax.ShapeDtypeStruct((N, F2), _F32)] * 2,
    )(d0, d1, x, W1)

    g0, g1 = agg(hs0, hs1, src2d, dst2d)

    q0, q1 = pl.pallas_call(
        _tc2_body,
        grid=grid,
        in_specs=[_row_spec(BLK, 1), _row_spec(BLK, 1),
                  _row_spec(BLK, F2), _row_spec(BLK, F2),
                  _full_spec((1, H)), _full_spec((H, H))],
        out_specs=[_row_spec(BLK, F2), _row_spec(BLK, F2)],
        out_shape=[jax.ShapeDtypeStruct((N, F2), _F32)] * 2,
    )(d0, d1, g0, g1, b1.reshape(1, H), W2)

    a0, a1 = agg(q0, q1, src2d, dst2d)

    out = pl.pallas_call(
        _tc3_body,
        grid=grid,
        in_specs=[_row_spec(BLK, 1), _row_spec(BLK, 1),
                  _row_spec(BLK, F2), _row_spec(BLK, F2),
                  _full_spec((1, H)), _full_spec((H, 1)), _full_spec((1, 1))],
        out_specs=_row_spec(BLK, 1),
        out_shape=jax.ShapeDtypeStruct((N, 1), _F32),
    )(d0, d1, a0, a1, b2.reshape(1, H), Wh, bh.reshape(1, 1))

    return out.reshape(N)


# speedup vs baseline: 1.2808x; 1.2808x over previous
import functools

import jax
import jax.numpy as jnp
from jax import lax
from jax.experimental import pallas as pl
from jax.experimental.pallas import tpu as pltpu
from jax.experimental.pallas import tpu_sc as plsc

_F32 = jnp.float32
_B = 125


def _node_chunks(N):
    r_main = ((N // 16 + 15) // 16) * 16
    r_last = N - 15 * r_main
    assert 0 < r_last <= r_main
    return r_main, r_last


def _chunk_copy(src, dst, s, r_main, r_last):
    @pl.when(s < 15)
    def _():
        pltpu.sync_copy(src.at[pl.ds(s * r_main, r_main)],
                        dst.at[pl.ds(s * r_main, r_main)])

    @pl.when(s == 15)
    def _():
        pltpu.sync_copy(src.at[pl.ds(15 * r_main, r_last)],
                        dst.at[pl.ds(15 * r_main, r_last)])


def _make_deg_kernel(N, E):
    EC = E // 32
    VR = EC // 16
    assert EC % 16 == 0
    r_main, r_last = _node_chunks(N)
    mesh = plsc.VectorSubcoreMesh(core_axis_name="c", subcore_axis_name="s")

    @functools.partial(
        pl.kernel,
        out_type=[jax.ShapeDtypeStruct((N,), _F32),
                  jax.ShapeDtypeStruct((N,), _F32)],
        mesh=mesh,
        scratch_types=[
            pltpu.VMEM((EC,), jnp.int32),
            pltpu.VMEM((N,), _F32),
            pltpu.VMEM((r_main,), _F32),
            pltpu.VMEM((r_main,), _F32),
            pltpu.VMEM_SHARED((16 * N,), _F32),
        ],
        compiler_params=pltpu.CompilerParams(needs_layout_passes=False),
    )
    def deg_kernel(dst1d, out0, out1, idxbuf, hist, tmp, accsum, spacc):
        c = lax.axis_index("c")
        s = lax.axis_index("s")
        w = c * 16 + s
        zero16 = jnp.zeros((16,), _F32)

        def fill_z(i, carry):
            hist[pl.ds(i * 16, 16)] = zero16
            return carry

        lax.fori_loop(0, N // 16, fill_z, 0)
        pltpu.sync_copy(dst1d.at[pl.ds(w * EC, EC)], idxbuf)

        def count(i, carry):
            idx = idxbuf[pl.ds(i * 16, 16)]
            cnt, last = plsc.scan_count(idx)
            plsc.addupdate_scatter(hist, [idx], cnt.astype(_F32), mask=last)
            return carry

        lax.fori_loop(0, VR, count, 0)
        pltpu.sync_copy(hist, spacc.at[pl.ds(s * N, N)])
        plsc.subcore_barrier()

        def reduce_write(r0, nr, out):
            pltpu.sync_copy(spacc.at[pl.ds(r0, nr)], accsum.at[pl.ds(0, nr)])

            def add_hist(j, carry):
                pltpu.sync_copy(spacc.at[pl.ds(j * N + r0, nr)],
                                tmp.at[pl.ds(0, nr)])

                def add_vec(v, carry2):
                    sl = pl.ds(v * 16, 16)
                    accsum[sl] = accsum[sl] + tmp[sl]
                    return carry2

                lax.fori_loop(0, nr // 16, add_vec, 0)
                return carry

            lax.fori_loop(1, 16, add_hist, 0)
            pltpu.sync_copy(accsum.at[pl.ds(0, nr)], out.at[pl.ds(r0, nr)])

        def write_core(out):
            @pl.when(s < 15)
            def _():
                reduce_write(s * r_main, r_main, out)

            @pl.when(s == 15)
            def _():
                reduce_write(15 * r_main, r_last, out)

        @pl.when(c == 0)
        def _():
            write_core(out0)

        @pl.when(c == 1)
        def _():
            write_core(out1)

    return deg_kernel


def _make_agg_kernel(N, E):
    n_rows = E // _B
    KB = n_rows // 16
    CH = 16
    assert KB % CH == 0 and KB * 16 == n_rows
    r_main, r_last = _node_chunks(N)
    mesh = plsc.VectorSubcoreMesh(core_axis_name="c", subcore_axis_name="s")

    @functools.partial(
        pl.kernel,
        out_type=[jax.ShapeDtypeStruct((N, 128), _F32),
                  jax.ShapeDtypeStruct((N, 128), _F32)],
        mesh=mesh,
        scratch_types=[
            pltpu.VMEM((CH, _B), jnp.int32),
            pltpu.VMEM((CH, _B), jnp.int32),
            pltpu.VMEM((_B, 128), _F32),
            pltpu.VMEM((_B, 128), _F32),
            pltpu.VMEM_SHARED((N, 128), _F32),
            pltpu.SemaphoreType.DMA,
            pltpu.SemaphoreType.DMA,
        ],
    )
    def agg_kernel(xs0, xs1, src2d, dst2d, out0, out1,
                   isrc, idst, buf0, buf1, acc, sem0, sem1):
        c = lax.axis_index("c")
        s = lax.axis_index("s")
        row0 = s * KB

        def run(xs, out):
            _chunk_copy(xs, acc, s, r_main, r_last)
            plsc.subcore_barrier()

            def chunk(t, carry):
                r = row0 + t * CH
                pltpu.sync_copy(src2d.at[pl.ds(r, CH)], isrc)
                pltpu.sync_copy(dst2d.at[pl.ds(r, CH)], idst)

                def pair(p, carry2):
                    @pl.when(p == 0)
                    def _():
                        pltpu.async_copy(xs.at[isrc.at[0]], buf0, sem0)

                    pltpu.async_copy(xs.at[isrc.at[2 * p + 1]], buf1, sem1)
                    pltpu.make_async_copy(xs.at[isrc.at[2 * p]], buf0,
                                          sem0).wait()
                    pltpu.sync_copy(buf0, acc.at[idst.at[2 * p]], add=True)

                    @pl.when(p < CH // 2 - 1)
                    def _():
                        pltpu.async_copy(xs.at[isrc.at[2 * p + 2]], buf0,
                                         sem0)

                    pltpu.make_async_copy(xs.at[isrc.at[2 * p + 1]], buf1,
                                          sem1).wait()
                    pltpu.sync_copy(buf1, acc.at[idst.at[2 * p + 1]],
                                    add=True)
                    return carry2

                lax.fori_loop(0, CH // 2, pair, 0)
                return carry

            lax.fori_loop(0, KB // CH, chunk, 0)
            plsc.subcore_barrier()
            _chunk_copy(acc, out, s, r_main, r_last)

        @pl.when(c == 0)
        def _():
            run(xs0, out0)

        @pl.when(c == 1)
        def _():
            run(xs1, out1)

    return agg_kernel


def _dis_of(d0_blk, d1_blk):
    return lax.rsqrt(d0_blk[...] + d1_blk[...] + 1.0)


def _tc1_body(d0, d1, x, w1, hs0, hs1):
    dis = _dis_of(d0, d1)
    h = jnp.dot(x[...], w1[...], preferred_element_type=_F32) * dis
    m = h.shape[1] // 2
    hs0[...] = h[:, :m]
    hs1[...] = h[:, m:]


def _tc2_body(d0, d1, g0, g1, b1, w2, hs0, hs1):
    dis = _dis_of(d0, d1)
    a = jnp.concatenate([g0[...], g1[...]], axis=1) * dis
    h1 = jnp.maximum(a + b1[...], 0.0)
    h = jnp.dot(h1, w2[...], preferred_element_type=_F32) * dis
    m = h.shape[1] // 2
    hs0[...] = h[:, :m]
    hs1[...] = h[:, m:]


def _tc3_body(d0, d1, a0, a1, b2, wh, bh, out):
    dis = _dis_of(d0, d1)
    a = jnp.concatenate([a0[...], a1[...]], axis=1) * dis
    h2 = jnp.maximum(a + b2[...], 0.0)
    out[...] = jnp.dot(h2, wh[...], preferred_element_type=_F32) + bh[...]


def _row_spec(blk, cols):
    return pl.BlockSpec((blk, cols), lambda i: (i, 0))


def _full_spec(shape):
    return pl.BlockSpec(shape, lambda i: (0,) * len(shape))


def kernel(x, edge_index, W1, b1, W2, b2, Wh, bh):
    N, D = x.shape
    H = W1.shape[1]
    E = edge_index.shape[1]
    F2 = H // 2
    BLK = 1000
    grid = (N // BLK,)

    src1d = edge_index[0].astype(jnp.int32)
    dst1d = edge_index[1].astype(jnp.int32)
    src2d = src1d.reshape(E // _B, _B)
    dst2d = dst1d.reshape(E // _B, _B)

    p0, p1 = _make_deg_kernel(N, E)(dst1d)
    d0 = p0.reshape(N, 1)
    d1 = p1.reshape(N, 1)

    agg = _make_agg_kernel(N, E)

    hs0, hs1 = pl.pallas_call(
        _tc1_body,
        grid=grid,
        in_specs=[_row_spec(BLK, 1), _row_spec(BLK, 1), _row_spec(BLK, D),
                  _full_spec((D, H))],
        out_specs=[_row_spec(BLK, F2), _row_spec(BLK, F2)],
        out_shape=[jax.ShapeDtypeStruct((N, F2), _F32)] * 2,
    )(d0, d1, x, W1)

    g0, g1 = agg(hs0, hs1, src2d, dst2d)

    q0, q1 = pl.pallas_call(
        _tc2_body,
        grid=grid,
        in_specs=[_row_spec(BLK, 1), _row_spec(BLK, 1),
                  _row_spec(BLK, F2), _row_spec(BLK, F2),
                  _full_spec((1, H)), _full_spec((H, H))],
        out_specs=[_row_spec(BLK, F2), _row_spec(BLK, F2)],
        out_shape=[jax.ShapeDtypeStruct((N, F2), _F32)] * 2,
    )(d0, d1, g0, g1, b1.reshape(1, H), W2)

    a0, a1 = agg(q0, q1, src2d, dst2d)

    out = pl.pallas_call(
        _tc3_body,
        grid=grid,
        in_specs=[_row_spec(BLK, 1), _row_spec(BLK, 1),
                  _row_spec(BLK, F2), _row_spec(BLK, F2),
                  _full_spec((1, H)), _full_spec((H, 1)), _full_spec((1, 1))],
        out_specs=_row_spec(BLK, 1),
        out_shape=jax.ShapeDtypeStruct((N, 1), _F32),
    )(d0, d1, a0, a1, b2.reshape(1, H), Wh, bh.reshape(1, 1))

    return out.reshape(N)


# speedup vs baseline: 1.3470x; 1.0517x over previous
import functools

import jax
import jax.numpy as jnp
from jax import lax
from jax.experimental import pallas as pl
from jax.experimental.pallas import tpu as pltpu
from jax.experimental.pallas import tpu_sc as plsc

_F32 = jnp.float32
_B = 125


def _node_chunks(N):
    r_main = ((N // 16 + 15) // 16) * 16
    r_last = N - 15 * r_main
    assert 0 < r_last <= r_main
    return r_main, r_last


def _chunk_copy(src, dst, s, r_main, r_last):
    @pl.when(s < 15)
    def _():
        pltpu.sync_copy(src.at[pl.ds(s * r_main, r_main)],
                        dst.at[pl.ds(s * r_main, r_main)])

    @pl.when(s == 15)
    def _():
        pltpu.sync_copy(src.at[pl.ds(15 * r_main, r_last)],
                        dst.at[pl.ds(15 * r_main, r_last)])


def _make_deg_kernel(N, E):
    EC = E // 32
    VR = EC // 16
    assert EC % 16 == 0
    r_main, r_last = _node_chunks(N)
    mesh = plsc.VectorSubcoreMesh(core_axis_name="c", subcore_axis_name="s")

    @functools.partial(
        pl.kernel,
        out_type=[jax.ShapeDtypeStruct((N,), _F32),
                  jax.ShapeDtypeStruct((N,), _F32)],
        mesh=mesh,
        scratch_types=[
            pltpu.VMEM((EC,), jnp.int32),
            pltpu.VMEM((N,), _F32),
            pltpu.VMEM((r_main,), _F32),
            pltpu.VMEM((r_main,), _F32),
            pltpu.VMEM_SHARED((16 * N,), _F32),
        ],
        compiler_params=pltpu.CompilerParams(needs_layout_passes=False),
    )
    def deg_kernel(dst1d, out0, out1, idxbuf, hist, tmp, accsum, spacc):
        c = lax.axis_index("c")
        s = lax.axis_index("s")
        w = c * 16 + s
        zero16 = jnp.zeros((16,), _F32)

        def fill_z(i, carry):
            hist[pl.ds(i * 16, 16)] = zero16
            return carry

        lax.fori_loop(0, N // 16, fill_z, 0)
        pltpu.sync_copy(dst1d.at[pl.ds(w * EC, EC)], idxbuf)

        def count(i, carry):
            idx = idxbuf[pl.ds(i * 16, 16)]
            cnt, last = plsc.scan_count(idx)
            plsc.addupdate_scatter(hist, [idx], cnt.astype(_F32), mask=last)
            return carry

        lax.fori_loop(0, VR, count, 0)
        pltpu.sync_copy(hist, spacc.at[pl.ds(s * N, N)])
        plsc.subcore_barrier()

        def reduce_write(r0, nr, out):
            pltpu.sync_copy(spacc.at[pl.ds(r0, nr)], accsum.at[pl.ds(0, nr)])

            def add_hist(j, carry):
                pltpu.sync_copy(spacc.at[pl.ds(j * N + r0, nr)],
                                tmp.at[pl.ds(0, nr)])

                def add_vec(v, carry2):
                    sl = pl.ds(v * 16, 16)
                    accsum[sl] = accsum[sl] + tmp[sl]
                    return carry2

                lax.fori_loop(0, nr // 16, add_vec, 0)
                return carry

            lax.fori_loop(1, 16, add_hist, 0)
            pltpu.sync_copy(accsum.at[pl.ds(0, nr)], out.at[pl.ds(r0, nr)])

        def write_core(out):
            @pl.when(s < 15)
            def _():
                reduce_write(s * r_main, r_main, out)

            @pl.when(s == 15)
            def _():
                reduce_write(15 * r_main, r_last, out)

        @pl.when(c == 0)
        def _():
            write_core(out0)

        @pl.when(c == 1)
        def _():
            write_core(out1)

    return deg_kernel


def _make_agg_kernel(N, E):
    n_rows = E // _B
    KB = n_rows // 16
    CH = 32
    assert KB % CH == 0 and KB * 16 == n_rows
    r_main, r_last = _node_chunks(N)
    mesh = plsc.VectorSubcoreMesh(core_axis_name="c", subcore_axis_name="s")

    @functools.partial(
        pl.kernel,
        out_type=[jax.ShapeDtypeStruct((N, 128), _F32),
                  jax.ShapeDtypeStruct((N, 128), _F32)],
        mesh=mesh,
        scratch_types=[
            pltpu.VMEM((CH, _B), jnp.int32),
            pltpu.VMEM((CH, _B), jnp.int32),
            pltpu.VMEM((_B, 128), _F32),
            pltpu.VMEM((_B, 128), _F32),
            pltpu.VMEM_SHARED((N, 128), _F32),
            pltpu.SemaphoreType.DMA,
            pltpu.SemaphoreType.DMA,
        ],
    )
    def agg_kernel(xs0, xs1, src2d, dst2d, out0, out1,
                   isrc, idst, buf0, buf1, acc, sem0, sem1):
        c = lax.axis_index("c")
        s = lax.axis_index("s")
        row0 = s * KB

        def run(xs, out):
            _chunk_copy(xs, acc, s, r_main, r_last)
            plsc.subcore_barrier()

            def chunk(t, carry):
                r = row0 + t * CH
                pltpu.sync_copy(src2d.at[pl.ds(r, CH)], isrc)
                pltpu.sync_copy(dst2d.at[pl.ds(r, CH)], idst)

                def pair(p, carry2):
                    @pl.when(p == 0)
                    def _():
                        pltpu.async_copy(xs.at[isrc.at[0]], buf0, sem0)

                    pltpu.async_copy(xs.at[isrc.at[2 * p + 1]], buf1, sem1)
                    pltpu.make_async_copy(xs.at[isrc.at[2 * p]], buf0,
                                          sem0).wait()
                    pltpu.sync_copy(buf0, acc.at[idst.at[2 * p]], add=True)

                    @pl.when(p < CH // 2 - 1)
                    def _():
                        pltpu.async_copy(xs.at[isrc.at[2 * p + 2]], buf0,
                                         sem0)

                    pltpu.make_async_copy(xs.at[isrc.at[2 * p + 1]], buf1,
                                          sem1).wait()
                    pltpu.sync_copy(buf1, acc.at[idst.at[2 * p + 1]],
                                    add=True)
                    return carry2

                lax.fori_loop(0, CH // 2, pair, 0)
                return carry

            lax.fori_loop(0, KB // CH, chunk, 0)
            plsc.subcore_barrier()
            _chunk_copy(acc, out, s, r_main, r_last)

        @pl.when(c == 0)
        def _():
            run(xs0, out0)

        @pl.when(c == 1)
        def _():
            run(xs1, out1)

    return agg_kernel


def _dis_of(d0_blk, d1_blk):
    return lax.rsqrt(d0_blk[...] + d1_blk[...] + 1.0)


def _tc1_body(d0, d1, x, w1, hs0, hs1):
    dis = _dis_of(d0, d1)
    h = jnp.dot(x[...], w1[...], preferred_element_type=_F32) * dis
    m = h.shape[1] // 2
    hs0[...] = h[:, :m]
    hs1[...] = h[:, m:]


def _tc2_body(d0, d1, g0, g1, b1, w2, hs0, hs1):
    dis = _dis_of(d0, d1)
    a = jnp.concatenate([g0[...], g1[...]], axis=1) * dis
    h1 = jnp.maximum(a + b1[...], 0.0)
    h = jnp.dot(h1, w2[...], preferred_element_type=_F32) * dis
    m = h.shape[1] // 2
    hs0[...] = h[:, :m]
    hs1[...] = h[:, m:]


def _tc3_body(d0, d1, a0, a1, b2, wh, bh, out):
    dis = _dis_of(d0, d1)
    a = jnp.concatenate([a0[...], a1[...]], axis=1) * dis
    h2 = jnp.maximum(a + b2[...], 0.0)
    out[...] = jnp.dot(h2, wh[...], preferred_element_type=_F32) + bh[...]


def _row_spec(blk, cols):
    return pl.BlockSpec((blk, cols), lambda i: (i, 0))


def _full_spec(shape):
    return pl.BlockSpec(shape, lambda i: (0,) * len(shape))


def kernel(x, edge_index, W1, b1, W2, b2, Wh, bh):
    N, D = x.shape
    H = W1.shape[1]
    E = edge_index.shape[1]
    F2 = H // 2
    BLK = 1000
    grid = (N // BLK,)

    src1d = edge_index[0].astype(jnp.int32)
    dst1d = edge_index[1].astype(jnp.int32)
    src2d = src1d.reshape(E // _B, _B)
    dst2d = dst1d.reshape(E // _B, _B)

    p0, p1 = _make_deg_kernel(N, E)(dst1d)
    d0 = p0.reshape(N, 1)
    d1 = p1.reshape(N, 1)

    agg = _make_agg_kernel(N, E)

    hs0, hs1 = pl.pallas_call(
        _tc1_body,
        grid=grid,
        in_specs=[_row_spec(BLK, 1), _row_spec(BLK, 1), _row_spec(BLK, D),
                  _full_spec((D, H))],
        out_specs=[_row_spec(BLK, F2), _row_spec(BLK, F2)],
        out_shape=[jax.ShapeDtypeStruct((N, F2), _F32)] * 2,
    )(d0, d1, x, W1)

    g0, g1 = agg(hs0, hs1, src2d, dst2d)

    q0, q1 = pl.pallas_call(
        _tc2_body,
        grid=grid,
        in_specs=[_row_spec(BLK, 1), _row_spec(BLK, 1),
                  _row_spec(BLK, F2), _row_spec(BLK, F2),
                  _full_spec((1, H)), _full_spec((H, H))],
        out_specs=[_row_spec(BLK, F2), _row_spec(BLK, F2)],
        out_shape=[jax.ShapeDtypeStruct((N, F2), _F32)] * 2,
    )(d0, d1, g0, g1, b1.reshape(1, H), W2)

    a0, a1 = agg(q0, q1, src2d, dst2d)

    out = pl.pallas_call(
        _tc3_body,
        grid=grid,
        in_specs=[_row_spec(BLK, 1), _row_spec(BLK, 1),
                  _row_spec(BLK, F2), _row_spec(BLK, F2),
                  _full_spec((1, H)), _full_spec((H, 1)), _full_spec((1, 1))],
        out_specs=_row_spec(BLK, 1),
        out_shape=jax.ShapeDtypeStruct((N, 1), _F32),
    )(d0, d1, a0, a1, b2.reshape(1, H), Wh, bh.reshape(1, 1))

    return out.reshape(N)


# speedup vs baseline: 1.3648x; 1.0132x over previous
import functools

import jax
import jax.numpy as jnp
from jax import lax
from jax.experimental import pallas as pl
from jax.experimental.pallas import tpu as pltpu
from jax.experimental.pallas import tpu_sc as plsc

_F32 = jnp.float32
_B = 125


def _node_chunks(N):
    r_main = ((N // 16 + 15) // 16) * 16
    r_last = N - 15 * r_main
    assert 0 < r_last <= r_main
    return r_main, r_last


def _chunk_copy(src, dst, s, r_main, r_last):
    @pl.when(s < 15)
    def _():
        pltpu.sync_copy(src.at[pl.ds(s * r_main, r_main)],
                        dst.at[pl.ds(s * r_main, r_main)])

    @pl.when(s == 15)
    def _():
        pltpu.sync_copy(src.at[pl.ds(15 * r_main, r_last)],
                        dst.at[pl.ds(15 * r_main, r_last)])


def _make_deg_kernel(N, E):
    EC = E // 32
    VR = EC // 16
    assert EC % 16 == 0
    r_main, r_last = _node_chunks(N)
    mesh = plsc.VectorSubcoreMesh(core_axis_name="c", subcore_axis_name="s")

    @functools.partial(
        pl.kernel,
        out_type=[jax.ShapeDtypeStruct((N,), _F32),
                  jax.ShapeDtypeStruct((N,), _F32)],
        mesh=mesh,
        scratch_types=[
            pltpu.VMEM((EC,), jnp.int32),
            pltpu.VMEM((N,), _F32),
            pltpu.VMEM((r_main,), _F32),
            pltpu.VMEM((r_main,), _F32),
            pltpu.VMEM_SHARED((16 * N,), _F32),
        ],
        compiler_params=pltpu.CompilerParams(needs_layout_passes=False),
    )
    def deg_kernel(dst1d, out0, out1, idxbuf, hist, tmp, accsum, spacc):
        c = lax.axis_index("c")
        s = lax.axis_index("s")
        w = c * 16 + s
        zero16 = jnp.zeros((16,), _F32)

        def fill_z(i, carry):
            hist[pl.ds(i * 16, 16)] = zero16
            return carry

        lax.fori_loop(0, N // 16, fill_z, 0)
        pltpu.sync_copy(dst1d.at[pl.ds(w * EC, EC)], idxbuf)

        def count(i, carry):
            idx = idxbuf[pl.ds(i * 16, 16)]
            cnt, last = plsc.scan_count(idx)
            plsc.addupdate_scatter(hist, [idx], cnt.astype(_F32), mask=last)
            return carry

        lax.fori_loop(0, VR, count, 0)
        pltpu.sync_copy(hist, spacc.at[pl.ds(s * N, N)])
        plsc.subcore_barrier()

        def reduce_write(r0, nr, out):
            pltpu.sync_copy(spacc.at[pl.ds(r0, nr)], accsum.at[pl.ds(0, nr)])

            def add_hist(j, carry):
                pltpu.sync_copy(spacc.at[pl.ds(j * N + r0, nr)],
                                tmp.at[pl.ds(0, nr)])

                def add_vec(v, carry2):
                    sl = pl.ds(v * 16, 16)
                    accsum[sl] = accsum[sl] + tmp[sl]
                    return carry2

                lax.fori_loop(0, nr // 16, add_vec, 0)
                return carry

            lax.fori_loop(1, 16, add_hist, 0)
            pltpu.sync_copy(accsum.at[pl.ds(0, nr)], out.at[pl.ds(r0, nr)])

        def write_core(out):
            @pl.when(s < 15)
            def _():
                reduce_write(s * r_main, r_main, out)

            @pl.when(s == 15)
            def _():
                reduce_write(15 * r_main, r_last, out)

        @pl.when(c == 0)
        def _():
            write_core(out0)

        @pl.when(c == 1)
        def _():
            write_core(out1)

    return deg_kernel


def _make_agg_kernel(N, E):
    n_rows = E // _B
    KB = n_rows // 16
    CH = 40
    assert KB % CH == 0 and KB * 16 == n_rows
    r_main, r_last = _node_chunks(N)
    mesh = plsc.VectorSubcoreMesh(core_axis_name="c", subcore_axis_name="s")

    @functools.partial(
        pl.kernel,
        out_type=[jax.ShapeDtypeStruct((N, 128), _F32),
                  jax.ShapeDtypeStruct((N, 128), _F32)],
        mesh=mesh,
        scratch_types=[
            pltpu.VMEM((CH, _B), jnp.int32),
            pltpu.VMEM((CH, _B), jnp.int32),
            pltpu.VMEM((_B, 128), _F32),
            pltpu.VMEM((_B, 128), _F32),
            pltpu.VMEM_SHARED((N, 128), _F32),
            pltpu.SemaphoreType.DMA,
            pltpu.SemaphoreType.DMA,
        ],
    )
    def agg_kernel(xs0, xs1, src2d, dst2d, out0, out1,
                   isrc, idst, buf0, buf1, acc, sem0, sem1):
        c = lax.axis_index("c")
        s = lax.axis_index("s")
        row0 = s * KB

        def run(xs, out):
            _chunk_copy(xs, acc, s, r_main, r_last)
            plsc.subcore_barrier()

            def chunk(t, carry):
                r = row0 + t * CH
                pltpu.sync_copy(src2d.at[pl.ds(r, CH)], isrc)
                pltpu.sync_copy(dst2d.at[pl.ds(r, CH)], idst)

                def pair(p, carry2):
                    @pl.when(p == 0)
                    def _():
                        pltpu.async_copy(xs.at[isrc.at[0]], buf0, sem0)

                    pltpu.async_copy(xs.at[isrc.at[2 * p + 1]], buf1, sem1)
                    pltpu.make_async_copy(xs.at[isrc.at[2 * p]], buf0,
                                          sem0).wait()
                    pltpu.sync_copy(buf0, acc.at[idst.at[2 * p]], add=True)

                    @pl.when(p < CH // 2 - 1)
                    def _():
                        pltpu.async_copy(xs.at[isrc.at[2 * p + 2]], buf0,
                                         sem0)

                    pltpu.make_async_copy(xs.at[isrc.at[2 * p + 1]], buf1,
                                          sem1).wait()
                    pltpu.sync_copy(buf1, acc.at[idst.at[2 * p + 1]],
                                    add=True)
                    return carry2

                lax.fori_loop(0, CH // 2, pair, 0)
                return carry

            lax.fori_loop(0, KB // CH, chunk, 0)
            plsc.subcore_barrier()
            _chunk_copy(acc, out, s, r_main, r_last)

        @pl.when(c == 0)
        def _():
            run(xs0, out0)

        @pl.when(c == 1)
        def _():
            run(xs1, out1)

    return agg_kernel


def _dis_of(d0_blk, d1_blk):
    return lax.rsqrt(d0_blk[...] + d1_blk[...] + 1.0)


def _tc1_body(d0, d1, x, w1, hs0, hs1):
    dis = _dis_of(d0, d1)
    h = jnp.dot(x[...], w1[...], preferred_element_type=_F32) * dis
    m = h.shape[1] // 2
    hs0[...] = h[:, :m]
    hs1[...] = h[:, m:]


def _tc2_body(d0, d1, g0, g1, b1, w2, hs0, hs1):
    dis = _dis_of(d0, d1)
    a = jnp.concatenate([g0[...], g1[...]], axis=1) * dis
    h1 = jnp.maximum(a + b1[...], 0.0)
    h = jnp.dot(h1, w2[...], preferred_element_type=_F32) * dis
    m = h.shape[1] // 2
    hs0[...] = h[:, :m]
    hs1[...] = h[:, m:]


def _tc3_body(d0, d1, a0, a1, b2, wh, bh, out):
    dis = _dis_of(d0, d1)
    a = jnp.concatenate([a0[...], a1[...]], axis=1) * dis
    h2 = jnp.maximum(a + b2[...], 0.0)
    out[...] = jnp.dot(h2, wh[...], preferred_element_type=_F32) + bh[...]


def _row_spec(blk, cols):
    return pl.BlockSpec((blk, cols), lambda i: (i, 0))


def _full_spec(shape):
    return pl.BlockSpec(shape, lambda i: (0,) * len(shape))


def kernel(x, edge_index, W1, b1, W2, b2, Wh, bh):
    N, D = x.shape
    H = W1.shape[1]
    E = edge_index.shape[1]
    F2 = H // 2
    BLK = 1000
    grid = (N // BLK,)

    src1d = edge_index[0].astype(jnp.int32)
    dst1d = edge_index[1].astype(jnp.int32)
    src2d = src1d.reshape(E // _B, _B)
    dst2d = dst1d.reshape(E // _B, _B)

    p0, p1 = _make_deg_kernel(N, E)(dst1d)
    d0 = p0.reshape(N, 1)
    d1 = p1.reshape(N, 1)

    agg = _make_agg_kernel(N, E)

    hs0, hs1 = pl.pallas_call(
        _tc1_body,
        grid=grid,
        in_specs=[_row_spec(BLK, 1), _row_spec(BLK, 1), _row_spec(BLK, D),
                  _full_spec((D, H))],
        out_specs=[_row_spec(BLK, F2), _row_spec(BLK, F2)],
        out_shape=[jax.ShapeDtypeStruct((N, F2), _F32)] * 2,
    )(d0, d1, x, W1)

    g0, g1 = agg(hs0, hs1, src2d, dst2d)

    q0, q1 = pl.pallas_call(
        _tc2_body,
        grid=grid,
        in_specs=[_row_spec(BLK, 1), _row_spec(BLK, 1),
                  _row_spec(BLK, F2), _row_spec(BLK, F2),
                  _full_spec((1, H)), _full_spec((H, H))],
        out_specs=[_row_spec(BLK, F2), _row_spec(BLK, F2)],
        out_shape=[jax.ShapeDtypeStruct((N, F2), _F32)] * 2,
    )(d0, d1, g0, g1, b1.reshape(1, H), W2)

    a0, a1 = agg(q0, q1, src2d, dst2d)

    out = pl.pallas_call(
        _tc3_body,
        grid=grid,
        in_specs=[_row_spec(BLK, 1), _row_spec(BLK, 1),
                  _row_spec(BLK, F2), _row_spec(BLK, F2),
                  _full_spec((1, H)), _full_spec((H, 1)), _full_spec((1, 1))],
        out_specs=_row_spec(BLK, 1),
        out_shape=jax.ShapeDtypeStruct((N, 1), _F32),
    )(d0, d1, a0, a1, b2.reshape(1, H), Wh, bh.reshape(1, 1))

    return out.reshape(N)
